# baseline (speedup 1.0000x reference)
import functools

import jax
import jax.numpy as jnp
from jax import lax
from jax.experimental import pallas as pl
from jax.experimental.pallas import tpu as pltpu
from jax.experimental.pallas import tpu_sc as plsc

B = 512
D = 128
L = 16
NC = 2
NS = 16
NW = NC * NS
RW = B // NW
CH = B // L
PROWS = 8
KPOS = 3
KNEG = 5
MARGIN = 1.0
BOUNDARY = 5.0
EPS_POS = 1e-07

_MESH = plsc.VectorSubcoreMesh(core_axis_name="c", subcore_axis_name="s",
                               num_cores=NC, num_subcores=NS)


def _iota16():
    return lax.broadcasted_iota(jnp.int32, (L,), 0)


def _splat_max(v):
    m = plsc.cummax(v)
    m = lax.rev(m, (0,))
    return plsc.cummax(m)


def _splat_at(v, selj, lowest):
    return _splat_max(jnp.where(selj, v, lowest))


def _prep_kernel(x_ref, tcol_ref, trow_ref, dtm_ref, pos_ref):
    x = x_ref[...]
    tcol = tcol_ref[...]
    trow = trow_ref[...]
    dot = lax.dot_general(x, x, (((1,), (1,)), ((), ())),
                          preferred_element_type=jnp.float32)
    rows = lax.broadcasted_iota(jnp.int32, (B, B), 0)
    cols = lax.broadcasted_iota(jnp.int32, (B, B), 1)
    eye = rows == cols
    diag_m = jnp.where(eye, dot, 0.0)
    sq_col = jnp.sum(diag_m, axis=1, keepdims=True)
    sq_row = jnp.sum(diag_m, axis=0, keepdims=True)

    same = tcol == trow
    inf = jnp.float32(jnp.inf)
    neg_inf = jnp.float32(-jnp.inf)

    dt = sq_col - 2.0 * dot + sq_row
    dt = jnp.maximum(dt, 0.0)
    distt = jnp.where(dt == 0.0, 0.0, jnp.sqrt(dt))
    dtm_ref[...] = jnp.where(same, inf, distt)

    d = sq_row - 2.0 * dot + sq_col
    d = jnp.maximum(d, 0.0)
    dist = jnp.where(d == 0.0, 0.0, jnp.sqrt(d))

    counts = jnp.sum(same.astype(jnp.float32), axis=1, keepdims=True)
    anchors = (counts < BOUNDARY).astype(jnp.float32)

    curp = jnp.where(same & ~eye, dist, neg_inf)
    prow = []
    for k in range(KPOS):
        pd = jnp.max(curp, axis=1, keepdims=True)
        prow.append(pd)
        if k < KPOS - 1:
            hit = jnp.where(curp == pd, cols, B)
            first = jnp.min(hit, axis=1, keepdims=True)
            curp = jnp.where(cols == first, neg_inf, curp)

    def to_row(v_col):
        return jnp.sum(jnp.where(eye, v_col + jnp.zeros((B, B), jnp.float32),
                                 0.0), axis=0, keepdims=True)

    zero_rows = jnp.zeros((PROWS - KPOS - 1, B), jnp.float32)
    pos_ref[...] = jnp.concatenate(
        [to_row(p) + MARGIN for p in prow] + [to_row(anchors), zero_rows],
        axis=0)


def _col_kernel(dtm_hbm, th_hbm, colblk, stage):
    wid = lax.axis_index("s") * NC + lax.axis_index("c")
    base = wid * RW
    pltpu.sync_copy(dtm_hbm.at[pl.ds(base, RW)], colblk)

    inf = jnp.float32(jnp.inf)
    neg_inf = jnp.float32(-jnp.inf)

    def one_col(j):
        a1 = jnp.full((L,), inf)
        a2 = jnp.full((L,), inf)
        a3 = jnp.full((L,), inf)
        a4 = jnp.full((L,), inf)
        a5 = jnp.full((L,), inf)
        for c in range(CH):
            new = colblk[j, pl.ds(c * L, L)]
            lo = jnp.minimum(a1, new); new = jnp.maximum(a1, new); a1 = lo
            lo = jnp.minimum(a2, new); new = jnp.maximum(a2, new); a2 = lo
            lo = jnp.minimum(a3, new); new = jnp.maximum(a3, new); a3 = lo
            lo = jnp.minimum(a4, new); new = jnp.maximum(a4, new); a4 = lo
            a5 = jnp.minimum(a5, new)
        r, _ = plsc.sort_key_val(a1, a1)
        for ai in (a2, a3, a4, a5):
            s, _ = plsc.sort_key_val(ai, ai)
            r = jnp.minimum(r, lax.rev(s, (0,)))
            r, _ = plsc.sort_key_val(r, r)
        return _splat_at(r, _iota16() == (KNEG - 1), neg_inf)

    th_all = jnp.full((L,), inf)
    for j in range(RW):
        th_all = jnp.where(_iota16() == j, one_col(j), th_all)
    stage[...] = th_all
    pltpu.sync_copy(stage, th_hbm.at[wid])


def _final_kernel(dtm_ref, pos_ref, th_ref, out_ref):
    dtm = dtm_ref[...]
    th = th_ref[...]
    anch = pos_ref[pl.ds(KPOS, 1), :]
    inf = jnp.float32(jnp.inf)
    m2 = (dtm <= th) & (dtm < inf) & (anch > 0.5)
    wf = m2.astype(jnp.float32)
    s_total = jnp.float32(0.0)
    n_total = jnp.float32(0.0)
    for k in range(KPOS):
        pk = pos_ref[pl.ds(k, 1), :]
        tr = jnp.maximum(jnp.where(pk > -inf, pk, 0.0) - dtm, 0.0)
        wk = wf * jnp.where(pk > -inf, 1.0, 0.0)
        s_total = s_total + jnp.sum(wk * tr)
        n_total = n_total + jnp.sum(wk * (tr > EPS_POS).astype(jnp.float32))
    out_ref[...] = (s_total / (n_total + EPS_POS)).reshape(1, 1)


_col_phase = functools.partial(
    pl.kernel,
    out_type=jax.ShapeDtypeStruct((NW, L), jnp.float32),
    mesh=_MESH,
    compiler_params=pltpu.CompilerParams(needs_layout_passes=False),
    scratch_types=[
        pltpu.VMEM((RW, B), jnp.float32),
        pltpu.VMEM((L,), jnp.float32),
    ],
)(_col_kernel)


def kernel(input, target):
    tcol = target.reshape(B, 1)
    trow = target.reshape(1, B)
    dtm, pos = pl.pallas_call(
        _prep_kernel,
        out_shape=(jax.ShapeDtypeStruct((B, B), jnp.float32),
                   jax.ShapeDtypeStruct((PROWS, B), jnp.float32)),
    )(input, tcol, trow)
    th = _col_phase(dtm)
    out = pl.pallas_call(
        _final_kernel,
        out_shape=jax.ShapeDtypeStruct((1, 1), jnp.float32),
    )(dtm, pos, th.reshape(B, 1))
    return out.reshape(())


# speedup vs baseline: 1.0924x; 1.0924x over previous
import functools

import jax
import jax.numpy as jnp
from jax import lax
from jax.experimental import pallas as pl
from jax.experimental.pallas import tpu as pltpu
from jax.experimental.pallas import tpu_sc as plsc

B = 512
D = 128
L = 16
NC = 2
NS = 16
NW = NC * NS
RW = B // NW
CH = B // L
PROWS = 8
KPOS = 3
KNEG = 5
MARGIN = 1.0
BOUNDARY = 5.0
EPS_POS = 1e-07

_MESH = plsc.VectorSubcoreMesh(core_axis_name="c", subcore_axis_name="s",
                               num_cores=NC, num_subcores=NS)


def _iota16():
    return lax.broadcasted_iota(jnp.int32, (L,), 0)


def _splat_max(v):
    m = plsc.cummax(v)
    m = lax.rev(m, (0,))
    return plsc.cummax(m)


def _splat_at(v, selj, lowest):
    return _splat_max(jnp.where(selj, v, lowest))


def _prep_kernel(x_ref, tcol_ref, trow_ref, dtm_ref, pos_ref):
    x = x_ref[...]
    tcol = tcol_ref[...]
    trow = trow_ref[...]
    dot = lax.dot_general(x, x, (((1,), (1,)), ((), ())),
                          preferred_element_type=jnp.float32)
    rows = lax.broadcasted_iota(jnp.int32, (B, B), 0)
    cols = lax.broadcasted_iota(jnp.int32, (B, B), 1)
    eye = rows == cols
    diag_m = jnp.where(eye, dot, 0.0)
    sq_col = jnp.sum(diag_m, axis=1, keepdims=True)
    sq_row = jnp.sum(diag_m, axis=0, keepdims=True)

    same = tcol == trow
    inf = jnp.float32(jnp.inf)
    neg_inf = jnp.float32(-jnp.inf)

    dt = sq_col - 2.0 * dot + sq_row
    dt = jnp.maximum(dt, 0.0)
    distt = jnp.where(dt == 0.0, 0.0, jnp.sqrt(dt))
    dtm_ref[...] = jnp.where(same, inf, distt)

    counts = jnp.sum(same.astype(jnp.float32), axis=1, keepdims=True)
    anchors = (counts < BOUNDARY).astype(jnp.float32)

    pmask = same & ~eye
    pmaskf = pmask.astype(jnp.float32)
    nvals = jnp.sum(pmaskf, axis=1, keepdims=True)
    pmax = jnp.max(jnp.where(pmask, distt, neg_inf), axis=1, keepdims=True)
    pmin = jnp.min(jnp.where(pmask, distt, inf), axis=1, keepdims=True)
    psum = jnp.sum(jnp.where(pmask, distt, 0.0), axis=1, keepdims=True)
    p0 = pmax
    p1 = jnp.where(nvals >= 3.0, psum - pmax - pmin, neg_inf)
    p2 = jnp.where(nvals >= 2.0, pmin, neg_inf)
    prow = [p0, p1, p2]

    def to_row(v_col):
        return jnp.sum(jnp.where(eye, v_col + jnp.zeros((B, B), jnp.float32),
                                 0.0), axis=0, keepdims=True)

    zero_rows = jnp.zeros((PROWS - KPOS - 1, B), jnp.float32)
    pos_ref[...] = jnp.concatenate(
        [to_row(p) + MARGIN for p in prow] + [to_row(anchors), zero_rows],
        axis=0)


def _col_kernel(dtm_hbm, th_hbm, colblk, stage):
    wid = lax.axis_index("s") * NC + lax.axis_index("c")
    base = wid * RW
    pltpu.sync_copy(dtm_hbm.at[pl.ds(base, RW)], colblk)

    inf = jnp.float32(jnp.inf)
    neg_inf = jnp.float32(-jnp.inf)

    def per_col(j, th_all):
        a1 = jnp.full((L,), inf)
        a2 = jnp.full((L,), inf)
        a3 = jnp.full((L,), inf)
        a4 = jnp.full((L,), inf)
        a5 = jnp.full((L,), inf)
        for c in range(CH):
            new = colblk[j, pl.ds(c * L, L)]
            lo = jnp.minimum(a1, new); new = jnp.maximum(a1, new); a1 = lo
            lo = jnp.minimum(a2, new); new = jnp.maximum(a2, new); a2 = lo
            lo = jnp.minimum(a3, new); new = jnp.maximum(a3, new); a3 = lo
            lo = jnp.minimum(a4, new); new = jnp.maximum(a4, new); a4 = lo
            a5 = jnp.minimum(a5, new)
        r, _ = plsc.sort_key_val(a1, a1)
        for ai in (a2, a3, a4, a5):
            s, _ = plsc.sort_key_val(ai, ai)
            r = jnp.minimum(r, lax.rev(s, (0,)))
            r, _ = plsc.sort_key_val(r, r)
        th = _splat_at(r, _iota16() == (KNEG - 1), neg_inf)
        return jnp.where(_iota16() == j, th, th_all)

    th_all = lax.fori_loop(0, RW, per_col, jnp.full((L,), inf))
    stage[...] = th_all
    pltpu.sync_copy(stage, th_hbm.at[wid])


def _final_kernel(dtm_ref, pos_ref, th_ref, out_ref):
    dtm = dtm_ref[...]
    th = th_ref[...]
    anch = pos_ref[pl.ds(KPOS, 1), :]
    inf = jnp.float32(jnp.inf)
    m2 = (dtm <= th) & (dtm < inf) & (anch > 0.5)
    wf = m2.astype(jnp.float32)
    s_total = jnp.float32(0.0)
    n_total = jnp.float32(0.0)
    for k in range(KPOS):
        pk = pos_ref[pl.ds(k, 1), :]
        tr = jnp.maximum(jnp.where(pk > -inf, pk, 0.0) - dtm, 0.0)
        wk = wf * jnp.where(pk > -inf, 1.0, 0.0)
        s_total = s_total + jnp.sum(wk * tr)
        n_total = n_total + jnp.sum(wk * (tr > EPS_POS).astype(jnp.float32))
    out_ref[...] = (s_total / (n_total + EPS_POS)).reshape(1, 1)


_col_phase = functools.partial(
    pl.kernel,
    out_type=jax.ShapeDtypeStruct((NW, L), jnp.float32),
    mesh=_MESH,
    compiler_params=pltpu.CompilerParams(needs_layout_passes=False),
    scratch_types=[
        pltpu.VMEM((RW, B), jnp.float32),
        pltpu.VMEM((L,), jnp.float32),
    ],
)(_col_kernel)


def kernel(input, target):
    tcol = target.reshape(B, 1)
    trow = target.reshape(1, B)
    dtm, pos = pl.pallas_call(
        _prep_kernel,
        out_shape=(jax.ShapeDtypeStruct((B, B), jnp.float32),
                   jax.ShapeDtypeStruct((PROWS, B), jnp.float32)),
    )(input, tcol, trow)
    th = _col_phase(dtm)
    out = pl.pallas_call(
        _final_kernel,
        out_shape=jax.ShapeDtypeStruct((1, 1), jnp.float32),
    )(dtm, pos, th.reshape(B, 1))
    return out.reshape(())
